# baseline (speedup 1.0000x reference)
import functools

import jax
import jax.numpy as jnp
from jax import lax
from jax.experimental import pallas as pl
from jax.experimental.pallas import tpu as pltpu
from jax.experimental.pallas import tpu_sc as plsc

NUM_CORES = 2
NUM_SUBCORES = 16
NUM_WORKERS = NUM_CORES * NUM_SUBCORES
CHUNK = 128
DEPTH = 8


def _make_gather(n_rows: int, d: int, n_per_w: int):
    n_chunks = n_per_w // CHUNK
    mesh = plsc.VectorSubcoreMesh(
        core_axis_name="c", subcore_axis_name="s",
        num_cores=NUM_CORES, num_subcores=NUM_SUBCORES)

    @functools.partial(
        pl.kernel,
        out_type=jax.ShapeDtypeStruct((n_rows, d), jnp.float32),
        mesh=mesh,
        scratch_types=[
            pltpu.VMEM((n_per_w,), jnp.int32),
            pltpu.VMEM((n_per_w, d), jnp.float32),
            pltpu.SemaphoreType.DMA,
        ],
        compiler_params=pltpu.CompilerParams(use_tc_tiling_on_sc=False),
    )
    def gather(table_hbm, idx_hbm, out_hbm, idx_v, rows_v, sem):
        wid = lax.axis_index("s") * NUM_CORES + lax.axis_index("c")
        base = wid * n_per_w
        pltpu.sync_copy(idx_hbm.at[pl.ds(base, n_per_w)], idx_v)

        @pl.loop(0, n_chunks // DEPTH)
        def _(g):
            goff = g * (DEPTH * CHUNK)
            descs = []
            for b in range(DEPTH):
                off = goff + b * CHUNK
                descs.append(pltpu.async_copy(
                    table_hbm.at[idx_v.at[pl.ds(off, CHUNK)]],
                    rows_v.at[pl.ds(off, CHUNK)],
                    sem,
                ))
            for desc in descs:
                desc.wait()

        pltpu.sync_copy(rows_v, out_hbm.at[pl.ds(base, n_per_w)])

    return gather


def kernel(x, tables):
    batch, num_fields = x.shape
    num_emb, d = tables.shape[1], tables.shape[2]
    x = jnp.where(x < 0, num_emb - 1, x)
    offs = (jnp.arange(num_fields, dtype=jnp.int32) * num_emb)[None, :]
    gidx = (x + offs).reshape(-1)
    one = jnp.where(x[0, 0] > jnp.int32(-2147483600), jnp.float32(1.0),
                    jnp.float32(2.0))
    table_flat = tables.reshape(num_fields * num_emb, d) * one

    n_rows = batch * num_fields
    n_per_w = n_rows // NUM_WORKERS
    out = _make_gather(n_rows, d, n_per_w)(table_flat, gidx)
    return out.reshape(batch, num_fields * d)


# speedup vs baseline: 1.8573x; 1.8573x over previous
import functools

import jax
import jax.numpy as jnp
from jax import lax
from jax.experimental import pallas as pl
from jax.experimental.pallas import tpu as pltpu
from jax.experimental.pallas import tpu_sc as plsc

NUM_CORES = 2
NUM_SUBCORES = 16
NUM_WORKERS = NUM_CORES * NUM_SUBCORES
CHUNK = 128
DEPTH = 13


def _make_gather(n_elem: int, n_per_w: int):
    n_chunks = n_per_w // CHUNK
    mesh = plsc.VectorSubcoreMesh(
        core_axis_name="c", subcore_axis_name="s",
        num_cores=NUM_CORES, num_subcores=NUM_SUBCORES)

    @functools.partial(
        pl.kernel,
        out_type=jax.ShapeDtypeStruct((n_elem,), jnp.float32),
        mesh=mesh,
        scratch_types=[
            pltpu.VMEM((n_per_w,), jnp.int32),
            pltpu.VMEM((n_per_w,), jnp.float32),
            pltpu.SemaphoreType.DMA,
        ],
        compiler_params=pltpu.CompilerParams(use_tc_tiling_on_sc=False),
    )
    def gather(table_hbm, eidx_hbm, out_hbm, ev, vals, sem):
        wid = lax.axis_index("s") * NUM_CORES + lax.axis_index("c")
        base = wid * n_per_w
        pltpu.sync_copy(eidx_hbm.at[pl.ds(base, n_per_w)], ev)

        @pl.loop(0, n_chunks // DEPTH)
        def _(g):
            goff = g * (DEPTH * CHUNK)
            descs = []
            for b in range(DEPTH):
                off = goff + b * CHUNK
                descs.append(pltpu.async_copy(
                    table_hbm.at[ev.at[pl.ds(off, CHUNK)]],
                    vals.at[pl.ds(off, CHUNK)],
                    sem,
                ))
            for desc in descs:
                desc.wait()

        pltpu.sync_copy(vals, out_hbm.at[pl.ds(base, n_per_w)])

    return gather


def kernel(x, tables):
    batch, num_fields = x.shape
    num_emb, d = tables.shape[1], tables.shape[2]
    x = jnp.where(x < 0, num_emb - 1, x)
    offs = (jnp.arange(num_fields, dtype=jnp.int32) * num_emb)[None, :]
    gidx = (x + offs).reshape(-1)
    eidx = (gidx[:, None] * d + jnp.arange(d, dtype=jnp.int32)).reshape(-1)
    table_flat = tables.reshape(-1)

    n_elem = batch * num_fields * d
    n_per_w = n_elem // NUM_WORKERS
    out = _make_gather(n_elem, n_per_w)(table_flat, eidx)
    return out.reshape(batch, num_fields * d)


# speedup vs baseline: 9.7303x; 5.2391x over previous
import functools

import jax
import jax.numpy as jnp
from jax import lax
from jax.experimental import pallas as pl
from jax.experimental.pallas import tpu as pltpu
from jax.experimental.pallas import tpu_sc as plsc

NUM_CORES = 2
NUM_SUBCORES = 16
NUM_WORKERS = NUM_CORES * NUM_SUBCORES
LANES = 16
CHUNK = 128
DEPTH = 13


def _make_gather(n_lookups: int, d: int, stride: int, n_per_w: int):
    n_elem_w = n_per_w * d
    n_chunks = n_elem_w // CHUNK
    mesh = plsc.VectorSubcoreMesh(
        core_axis_name="c", subcore_axis_name="s",
        num_cores=NUM_CORES, num_subcores=NUM_SUBCORES)

    @functools.partial(
        pl.kernel,
        out_type=jax.ShapeDtypeStruct((n_lookups * d,), jnp.float32),
        mesh=mesh,
        scratch_types=[
            pltpu.VMEM((n_per_w,), jnp.int32),
            pltpu.VMEM((n_elem_w,), jnp.int32),
            pltpu.VMEM((n_elem_w,), jnp.float32),
            pltpu.SemaphoreType.DMA,
        ],
        compiler_params=pltpu.CompilerParams(
            use_tc_tiling_on_sc=False, needs_layout_passes=False),
    )
    def gather(table_hbm, base_hbm, out_hbm, ev0, ev, vals, sem):
        wid = lax.axis_index("s") * NUM_CORES + lax.axis_index("c")
        base = wid * n_per_w
        pltpu.sync_copy(base_hbm.at[pl.ds(base, n_per_w)], ev0)

        lane = lax.iota(jnp.int32, LANES)

        @pl.loop(0, n_per_w // LANES)
        def _(j):
            e = ev0[pl.ds(j * LANES, LANES)]
            pos = j * (LANES * d) + lane * d
            for q in range(d):
                plsc.store_scatter(ev, [pos + q], e + q * stride)

        @pl.loop(0, n_chunks // DEPTH)
        def _(g):
            goff = g * (DEPTH * CHUNK)
            descs = []
            for b in range(DEPTH):
                off = goff + b * CHUNK
                descs.append(pltpu.async_copy(
                    table_hbm.at[ev.at[pl.ds(off, CHUNK)]],
                    vals.at[pl.ds(off, CHUNK)],
                    sem,
                ))
            for desc in descs:
                desc.wait()

        pltpu.sync_copy(vals, out_hbm.at[pl.ds(base * d, n_elem_w)])

    return gather


def kernel(x, tables):
    batch, num_fields = x.shape
    num_emb, d = tables.shape[1], tables.shape[2]
    x = jnp.where(x < 0, num_emb - 1, x)
    x1 = x.reshape(-1)
    k = jnp.arange(batch * num_fields, dtype=jnp.int32)
    base_idx = x1 + (k % num_fields) * (d * num_emb)
    table_flat = tables.transpose(0, 2, 1).reshape(-1)

    n_lookups = batch * num_fields
    n_per_w = n_lookups // NUM_WORKERS
    out = _make_gather(n_lookups, d, num_emb, n_per_w)(table_flat, base_idx)
    return out.reshape(batch, num_fields * d)


# speedup vs baseline: 24.0267x; 2.4693x over previous
import functools

import jax
import jax.numpy as jnp
from jax import lax
from jax.experimental import pallas as pl
from jax.experimental.pallas import tpu as pltpu
from jax.experimental.pallas import tpu_sc as plsc

NUM_CORES = 2
NUM_SUBCORES = 16
NUM_WORKERS = NUM_CORES * NUM_SUBCORES
LANES = 16
CHUNK = 128
DEPTH = 13


def _make_gather(n_lookups: int, d: int, stride: int, n_per_w: int):
    n_elem_w = n_per_w * d
    n_chunks = n_elem_w // CHUNK
    mesh = plsc.VectorSubcoreMesh(
        core_axis_name="c", subcore_axis_name="s",
        num_cores=NUM_CORES, num_subcores=NUM_SUBCORES)

    @functools.partial(
        pl.kernel,
        out_type=jax.ShapeDtypeStruct((n_lookups * d,), jnp.float32),
        mesh=mesh,
        scratch_types=[
            pltpu.VMEM((n_per_w,), jnp.int32),
            pltpu.VMEM((n_elem_w,), jnp.int32),
            pltpu.VMEM((n_elem_w,), jnp.float32),
            pltpu.SemaphoreType.DMA,
        ],
        compiler_params=pltpu.CompilerParams(
            use_tc_tiling_on_sc=False, needs_layout_passes=False),
    )
    def gather(table_hbm, base_hbm, out_hbm, ev0, ev, vals, sem):
        wid = lax.axis_index("s") * NUM_CORES + lax.axis_index("c")
        base = wid * n_per_w
        pltpu.sync_copy(base_hbm.at[pl.ds(base, n_per_w)], ev0)

        lane = lax.iota(jnp.int32, LANES)

        @pl.loop(0, n_per_w // LANES)
        def _(j):
            e = ev0[pl.ds(j * LANES, LANES)]
            pos = j * (LANES * d) + lane * d
            for q in range(d):
                plsc.store_scatter(ev, [pos + q], e + q * stride)

        @pl.loop(0, n_chunks // DEPTH)
        def _(g):
            goff = g * (DEPTH * CHUNK)
            descs = []
            for b in range(DEPTH):
                off = goff + b * CHUNK
                descs.append(pltpu.async_copy(
                    table_hbm.at[ev.at[pl.ds(off, CHUNK)]],
                    vals.at[pl.ds(off, CHUNK)],
                    sem,
                ))
            for desc in descs:
                desc.wait()

        pltpu.sync_copy(vals, out_hbm.at[pl.ds(base * d, n_elem_w)])

    return gather


def kernel(x, tables):
    batch, num_fields = x.shape
    num_emb, d = tables.shape[1], tables.shape[2]
    x = jnp.where(x < 0, num_emb - 1, x)
    x1 = x.reshape(-1)

    v_pad = -num_emb % 128
    vt = (num_emb + v_pad) // 128
    p = jnp.pad(tables, ((0, 0), (0, v_pad), (0, 0)))
    table_flat = (
        p.reshape(num_fields, vt, 128, d).transpose(0, 1, 3, 2).reshape(-1))

    k = jnp.arange(batch * num_fields, dtype=jnp.int32)
    base_idx = ((k % num_fields) * (vt * d * 128)
                + (x1 >> 7) * (d * 128) + (x1 & 127))

    n_lookups = batch * num_fields
    n_per_w = n_lookups // NUM_WORKERS
    out = _make_gather(n_lookups, d, 128, n_per_w)(table_flat, base_idx)
    return out.reshape(batch, num_fields * d)


# speedup vs baseline: 24.0391x; 1.0005x over previous
import functools

import jax
import jax.numpy as jnp
from jax import lax
from jax.experimental import pallas as pl
from jax.experimental.pallas import tpu as pltpu
from jax.experimental.pallas import tpu_sc as plsc

NUM_CORES = 2
NUM_SUBCORES = 16
NUM_WORKERS = NUM_CORES * NUM_SUBCORES
LANES = 16
CHUNK = 128
DEPTH = 13


def _make_gather(n_lookups: int, d: int, stride: int, n_per_w: int):
    n_elem_w = n_per_w * d
    n_chunks = n_elem_w // CHUNK
    mesh = plsc.VectorSubcoreMesh(
        core_axis_name="c", subcore_axis_name="s",
        num_cores=NUM_CORES, num_subcores=NUM_SUBCORES)

    @functools.partial(
        pl.kernel,
        out_type=jax.ShapeDtypeStruct((n_lookups * d,), jnp.float32),
        mesh=mesh,
        scratch_types=[
            pltpu.VMEM((n_per_w,), jnp.int32),
            pltpu.VMEM((n_elem_w,), jnp.int32),
            pltpu.VMEM((n_elem_w,), jnp.float32),
            pltpu.SemaphoreType.DMA,
        ],
        compiler_params=pltpu.CompilerParams(
            use_tc_tiling_on_sc=False, needs_layout_passes=False),
    )
    def gather(table_hbm, base_hbm, out_hbm, ev0, ev, vals, sem):
        wid = lax.axis_index("s") * NUM_CORES + lax.axis_index("c")
        base = wid * n_per_w
        pltpu.sync_copy(base_hbm.at[pl.ds(base, n_per_w)], ev0)

        lane = lax.iota(jnp.int32, LANES)

        @pl.loop(0, n_per_w // LANES)
        def _(j):
            e = ev0[pl.ds(j * LANES, LANES)]
            pos = j * (LANES * d) + lane * d
            for q in range(d):
                plsc.store_scatter(ev, [pos + q], e + q * stride)

        @pl.loop(0, n_chunks // DEPTH)
        def _(g):
            goff = g * (DEPTH * CHUNK)
            descs = []
            for b in range(DEPTH):
                off = goff + b * CHUNK
                descs.append(pltpu.async_copy(
                    table_hbm.at[ev.at[pl.ds(off, CHUNK)]],
                    vals.at[pl.ds(off, CHUNK)],
                    sem,
                ))
            for desc in descs:
                desc.wait()

        pltpu.sync_copy(vals, out_hbm.at[pl.ds(base * d, n_elem_w)])

    return gather


def kernel(x, tables):
    batch, num_fields = x.shape
    num_emb, d = tables.shape[1], tables.shape[2]
    x = jnp.where(x < 0, num_emb - 1, x)
    x1 = x.reshape(-1)

    v_pad = -num_emb % 128
    vt = (num_emb + v_pad) // 128
    p = jnp.pad(tables.transpose(0, 2, 1), ((0, 0), (0, 0), (0, v_pad)))
    table_flat = (
        p.reshape(num_fields, d, vt, 128).transpose(0, 2, 1, 3).reshape(-1))

    k = jnp.arange(batch * num_fields, dtype=jnp.int32)
    base_idx = ((k % num_fields) * (vt * d * 128)
                + (x1 >> 7) * (d * 128) + (x1 & 127))

    n_lookups = batch * num_fields
    n_per_w = n_lookups // NUM_WORKERS
    out = _make_gather(n_lookups, d, 128, n_per_w)(table_flat, base_idx)
    return out.reshape(batch, num_fields * d)
